# baseline (speedup 1.0000x reference)
import functools

import jax
import jax.numpy as jnp
from jax import lax
from jax.experimental import pallas as pl
from jax.experimental.pallas import tpu as pltpu
from jax.experimental.pallas import tpu_sc as plsc

N = 10000
D = 128
E = 320000
NLAB = 1440
NC = 2
NS = 16
HALFN = 5120
CHUNK = 64
CPT = 320
GC = 8
E_PAD = CHUNK * CPT * NS
ART = HALFN // NS
LANES = 16
RB = 32
SENT = -1


def _seg_sum_body(h_hbm, src_hbm, dst_hbm, out_hbm,
                  rsrc_v, rdst_v, fsrc_v, fdst_v, rows_v, table_sh, agg_sh):
    cid = lax.axis_index("c")
    sid = lax.axis_index("s")
    base = cid * HALFN

    def _zb(t, carry):
        rows_v[t // (D // LANES),
               pl.ds((t % (D // LANES)) * LANES, LANES)] = (
            jnp.zeros((LANES,), jnp.float32))
        return carry
    lax.fori_loop(0, RB * (D // LANES), _zb, 0)
    for k in range(ART // RB):
        pltpu.sync_copy(rows_v, agg_sh.at[pl.ds(sid * ART + k * RB, RB)])

    @pl.when(sid < NS - 1)
    def _():
        for k in range(640 // RB):
            sl = pl.ds(sid * 640 + k * RB, RB)
            pltpu.sync_copy(h_hbm.at[sl], rows_v)
            pltpu.sync_copy(rows_v, table_sh.at[sl])

    @pl.when(sid == NS - 1)
    def _():
        for k in range(400 // RB):
            sl = pl.ds(9600 + k * RB, RB)
            pltpu.sync_copy(h_hbm.at[sl], rows_v)
            pltpu.sync_copy(rows_v, table_sh.at[sl])
        tl = pl.ds(9984, 16)
        pltpu.sync_copy(h_hbm.at[tl], rows_v.at[pl.ds(0, 16)])
        pltpu.sync_copy(rows_v.at[pl.ds(0, 16)], table_sh.at[tl])

    plsc.subcore_barrier()

    def _group(j, carry):
        pltpu.sync_copy(src_hbm.at[pl.ds(sid * CPT + j * GC, GC)], rsrc_v)
        pltpu.sync_copy(dst_hbm.at[pl.ds(sid * CPT + j * GC, GC)], rdst_v)

        def _chunk(cg, carry2):
            for g in range(CHUNK // LANES):
                vs = rsrc_v[cg, pl.ds(g * LANES, LANES)]
                vd = rdst_v[cg, pl.ds(g * LANES, LANES)]
                loc = vd - base
                ok = jnp.logical_and(loc >= 0, loc < HALFN)
                fdst_v[g // 2, pl.ds((g % 2) * LANES, LANES)] = (
                    jnp.where(ok, loc, SENT))
                fsrc_v[g // 2, pl.ds((g % 2) * LANES, LANES)] = (
                    jnp.where(ok, vs, SENT))
            for hf in range(2):
                pltpu.sync_copy(
                    table_sh.at[plsc.Indices(fsrc_v.at[hf],
                                             ignored_value=SENT)], rows_v)
                pltpu.sync_copy(
                    rows_v,
                    agg_sh.at[plsc.Indices(fdst_v.at[hf],
                                           ignored_value=SENT)],
                    add=True)
            return carry2
        lax.fori_loop(0, GC, _chunk, 0)
        return carry
    lax.fori_loop(0, CPT // GC, _group, 0)
    plsc.subcore_barrier()

    for k in range(ART // RB):
        sl_a = pl.ds(sid * ART + k * RB, RB)
        sl_o = pl.ds(base + sid * ART + k * RB, RB)
        pltpu.sync_copy(agg_sh.at[sl_a], out_hbm.at[sl_o])


@functools.cache
def _seg_sum():
    mesh = plsc.VectorSubcoreMesh(
        core_axis_name="c", subcore_axis_name="s",
        num_cores=NC, num_subcores=NS)
    return pl.kernel(
        _seg_sum_body,
        out_type=jax.ShapeDtypeStruct((2 * HALFN, D), jnp.float32),
        mesh=mesh,
        scratch_types=[
            pltpu.VMEM((GC, CHUNK), jnp.int32),
            pltpu.VMEM((GC, CHUNK), jnp.int32),
            pltpu.VMEM((2, RB), jnp.int32),
            pltpu.VMEM((2, RB), jnp.int32),
            pltpu.VMEM((RB, D), jnp.float32),
            pltpu.VMEM_SHARED((N, D), jnp.float32),
            pltpu.VMEM_SHARED((HALFN, D), jnp.float32),
        ],
    )


BR = 1000


def _combine_body(a, h, w1, w2, b, o):
    acc = jnp.dot(a[...], w1[...], preferred_element_type=jnp.float32)
    acc = acc + jnp.dot(h[...], w2[...], preferred_element_type=jnp.float32)
    o[...] = jnp.maximum(acc + b[...], 0.0)


def _combine(a, h, w1, w2, b):
    return pl.pallas_call(
        _combine_body,
        grid=(N // BR,),
        in_specs=[
            pl.BlockSpec((BR, D), lambda i: (i, 0)),
            pl.BlockSpec((BR, D), lambda i: (i, 0)),
            pl.BlockSpec((D, D), lambda i: (0, 0)),
            pl.BlockSpec((D, D), lambda i: (0, 0)),
            pl.BlockSpec((1, D), lambda i: (0, 0)),
        ],
        out_specs=pl.BlockSpec((BR, D), lambda i: (i, 0)),
        out_shape=jax.ShapeDtypeStruct((N, D), jnp.float32),
    )(a, h, w1, w2, b)


def _dense_body(h, wd, bd, o):
    o[...] = jnp.dot(h[...], wd[...], preferred_element_type=jnp.float32) + bd[...]


def _dense(h, wd, bd):
    return pl.pallas_call(
        _dense_body,
        grid=(N // BR,),
        in_specs=[
            pl.BlockSpec((BR, D), lambda i: (i, 0)),
            pl.BlockSpec((D, NLAB), lambda i: (0, 0)),
            pl.BlockSpec((1, NLAB), lambda i: (0, 0)),
        ],
        out_specs=pl.BlockSpec((BR, NLAB), lambda i: (i, 0)),
        out_shape=jax.ShapeDtypeStruct((N, NLAB), jnp.float32),
    )(h, wd, bd)


def kernel(x, edge_index,
           W1_0, W2_0, b_0,
           W1_1, W2_1, b_1,
           W1_2, W2_2, b_2,
           W1_3, W2_3, b_3,
           W1_4, W2_4, b_4,
           W1_5, W2_5, b_5,
           W1_6, W2_6, b_6,
           Wd, bd):
    src = edge_index[0]
    dst = edge_index[1]
    src_p = jnp.concatenate(
        [src, jnp.zeros((E_PAD - E,), jnp.int32)]).reshape(
            E_PAD // CHUNK, CHUNK)
    dst_p = jnp.concatenate(
        [dst, jnp.full((E_PAD - E,), N, jnp.int32)]).reshape(
            E_PAD // CHUNK, CHUNK)

    layers = [
        (W1_0, W2_0, b_0), (W1_1, W2_1, b_1), (W1_2, W2_2, b_2),
        (W1_3, W2_3, b_3), (W1_4, W2_4, b_4), (W1_5, W2_5, b_5),
        (W1_6, W2_6, b_6),
    ]
    h = x
    for w1, w2, b in layers:
        a = _seg_sum()(h, src_p, dst_p)
        h = _combine(a, h, w1, w2, b.reshape(1, D))
    return _dense(h, Wd, bd.reshape(1, NLAB))


# speedup vs baseline: 1.2437x; 1.2437x over previous
import functools

import jax
import jax.numpy as jnp
from jax import lax
from jax.experimental import pallas as pl
from jax.experimental.pallas import tpu as pltpu
from jax.experimental.pallas import tpu_sc as plsc

N = 10000
D = 128
E = 320000
NLAB = 1440
NC = 2
NS = 16
HALFN = 5008
ARTF = 320
CHUNK = 32
CPT = 640
GC = 8
E_PAD = CHUNK * CPT * NS
LANES = 16
RB = 32
SENT = -1


def _seg_sum_body(h_hbm, src_hbm, dst_hbm, out_hbm,
                  eidx_v, rowsab_v, table_sh, agg_sh,
                  semg0, semg1):
    rows_v = rowsab_v.at[pl.ds(0, RB)]
    rowsb_v = rowsab_v.at[pl.ds(RB, RB)]
    cid = lax.axis_index("c")
    sid = lax.axis_index("s")
    base = cid * HALFN

    def _zb(t, carry):
        rows_v[t // (D // LANES),
               pl.ds((t % (D // LANES)) * LANES, LANES)] = (
            jnp.zeros((LANES,), jnp.float32))
        return carry
    lax.fori_loop(0, RB * (D // LANES), _zb, 0)

    @pl.when(sid < NS - 1)
    def _():
        for k in range(ARTF // RB):
            pltpu.sync_copy(rows_v,
                            agg_sh.at[pl.ds(sid * ARTF + k * RB, RB)])

    @pl.when(sid == NS - 1)
    def _():
        for k in range(6):
            pltpu.sync_copy(rows_v, agg_sh.at[pl.ds(4800 + k * RB, RB)])
        pltpu.sync_copy(rows_v.at[pl.ds(0, 16)], agg_sh.at[pl.ds(4992, 16)])

    @pl.when(sid < NS - 1)
    def _():
        for k in range(640 // RB):
            sl = pl.ds(sid * 640 + k * RB, RB)
            pltpu.sync_copy(h_hbm.at[sl], rows_v)
            pltpu.sync_copy(rows_v, table_sh.at[sl])

    @pl.when(sid == NS - 1)
    def _():
        for k in range(400 // RB):
            sl = pl.ds(9600 + k * RB, RB)
            pltpu.sync_copy(h_hbm.at[sl], rows_v)
            pltpu.sync_copy(rows_v, table_sh.at[sl])
        tl = pl.ds(9984, 16)
        pltpu.sync_copy(h_hbm.at[tl], rows_v.at[pl.ds(0, 16)])
        pltpu.sync_copy(rows_v.at[pl.ds(0, 16)], table_sh.at[tl])

    plsc.subcore_barrier()

    def _group(j, carry):
        pltpu.sync_copy(src_hbm.at[pl.ds(sid * CPT + j * GC, GC)],
                        eidx_v.at[pl.ds(0, GC)])
        pltpu.sync_copy(dst_hbm.at[pl.ds(sid * CPT + j * GC, GC)],
                        eidx_v.at[pl.ds(GC, GC)])
        for t in range(GC * (CHUNK // LANES)):
            row, col = t // (CHUNK // LANES), (t % (CHUNK // LANES)) * LANES
            vs = eidx_v[row, pl.ds(col, LANES)]
            vd = eidx_v[GC + row, pl.ds(col, LANES)]
            loc = vd - base
            ok = jnp.logical_and(loc >= 0, loc < HALFN)
            eidx_v[GC + row, pl.ds(col, LANES)] = jnp.where(ok, loc, SENT)
            eidx_v[row, pl.ds(col, LANES)] = jnp.where(ok, vs, SENT)

        def _gather(cg, buf, sem):
            return pltpu.async_copy(
                table_sh.at[plsc.Indices(eidx_v.at[cg], ignored_value=SENT)],
                buf, sem)

        def _scatter(cg, buf):
            pltpu.sync_copy(
                buf, agg_sh.at[plsc.Indices(eidx_v.at[GC + cg],
                                            ignored_value=SENT)],
                add=True)

        _gather(0, rows_v, semg0)
        for p in range(GC // 2):
            pltpu.make_async_copy(table_sh, rows_v, semg0).wait()
            if 2 * p + 1 < GC:
                _gather(2 * p + 1, rowsb_v, semg1)
            _scatter(2 * p, rows_v)
            if 2 * p + 1 < GC:
                pltpu.make_async_copy(table_sh, rowsb_v, semg1).wait()
                if 2 * p + 2 < GC:
                    _gather(2 * p + 2, rows_v, semg0)
                _scatter(2 * p + 1, rowsb_v)
        return carry
    lax.fori_loop(0, CPT // GC, _group, 0)
    plsc.subcore_barrier()

    @pl.when(sid < NS - 1)
    def _():
        for k in range(ARTF // RB):
            sl = sid * ARTF + k * RB
            pltpu.sync_copy(agg_sh.at[pl.ds(sl, RB)],
                            out_hbm.at[pl.ds(base + sl, RB)])

    @pl.when(sid == NS - 1)
    def _():
        for k in range(6):
            sl = 4800 + k * RB
            pltpu.sync_copy(agg_sh.at[pl.ds(sl, RB)],
                            out_hbm.at[pl.ds(base + sl, RB)])
        pltpu.sync_copy(agg_sh.at[pl.ds(4992, 16)],
                        out_hbm.at[pl.ds(base + 4992, 16)])


@functools.cache
def _seg_sum():
    mesh = plsc.VectorSubcoreMesh(
        core_axis_name="c", subcore_axis_name="s",
        num_cores=NC, num_subcores=NS)
    return pl.kernel(
        _seg_sum_body,
        out_type=jax.ShapeDtypeStruct((2 * HALFN, D), jnp.float32),
        mesh=mesh,
        scratch_types=[
            pltpu.VMEM((2 * GC, CHUNK), jnp.int32),
            pltpu.VMEM((2 * RB, D), jnp.float32),
            pltpu.VMEM_SHARED((N, D), jnp.float32),
            pltpu.VMEM_SHARED((HALFN, D), jnp.float32),
            pltpu.SemaphoreType.DMA,
            pltpu.SemaphoreType.DMA,
        ],
    )


BR = 1000


def _combine_body(a, h, w1, w2, b, o):
    acc = jnp.dot(a[...], w1[...], preferred_element_type=jnp.float32)
    acc = acc + jnp.dot(h[...], w2[...], preferred_element_type=jnp.float32)
    o[...] = jnp.maximum(acc + b[...], 0.0)


def _combine(a, h, w1, w2, b):
    return pl.pallas_call(
        _combine_body,
        grid=(N // BR,),
        in_specs=[
            pl.BlockSpec((BR, D), lambda i: (i, 0)),
            pl.BlockSpec((BR, D), lambda i: (i, 0)),
            pl.BlockSpec((D, D), lambda i: (0, 0)),
            pl.BlockSpec((D, D), lambda i: (0, 0)),
            pl.BlockSpec((1, D), lambda i: (0, 0)),
        ],
        out_specs=pl.BlockSpec((BR, D), lambda i: (i, 0)),
        out_shape=jax.ShapeDtypeStruct((N, D), jnp.float32),
    )(a, h, w1, w2, b)


def _dense_body(h, wd, bd, o):
    o[...] = jnp.dot(h[...], wd[...], preferred_element_type=jnp.float32) + bd[...]


def _dense(h, wd, bd):
    return pl.pallas_call(
        _dense_body,
        grid=(N // BR,),
        in_specs=[
            pl.BlockSpec((BR, D), lambda i: (i, 0)),
            pl.BlockSpec((D, NLAB), lambda i: (0, 0)),
            pl.BlockSpec((1, NLAB), lambda i: (0, 0)),
        ],
        out_specs=pl.BlockSpec((BR, NLAB), lambda i: (i, 0)),
        out_shape=jax.ShapeDtypeStruct((N, NLAB), jnp.float32),
    )(h, wd, bd)


def kernel(x, edge_index,
           W1_0, W2_0, b_0,
           W1_1, W2_1, b_1,
           W1_2, W2_2, b_2,
           W1_3, W2_3, b_3,
           W1_4, W2_4, b_4,
           W1_5, W2_5, b_5,
           W1_6, W2_6, b_6,
           Wd, bd):
    src = edge_index[0]
    dst = edge_index[1]
    src_p = jnp.concatenate(
        [src, jnp.zeros((E_PAD - E,), jnp.int32)]).reshape(
            E_PAD // CHUNK, CHUNK)
    dst_p = jnp.concatenate(
        [dst, jnp.full((E_PAD - E,), N, jnp.int32)]).reshape(
            E_PAD // CHUNK, CHUNK)

    layers = [
        (W1_0, W2_0, b_0), (W1_1, W2_1, b_1), (W1_2, W2_2, b_2),
        (W1_3, W2_3, b_3), (W1_4, W2_4, b_4), (W1_5, W2_5, b_5),
        (W1_6, W2_6, b_6),
    ]
    h = x
    for w1, w2, b in layers:
        a = _seg_sum()(h, src_p, dst_p)
        h = _combine(a, h, w1, w2, b.reshape(1, D))
    return _dense(h, Wd, bd.reshape(1, NLAB))


# speedup vs baseline: 1.2583x; 1.0117x over previous
import functools

import jax
import jax.numpy as jnp
from jax import lax
from jax.experimental import pallas as pl
from jax.experimental.pallas import tpu as pltpu
from jax.experimental.pallas import tpu_sc as plsc

N = 10000
D = 128
E = 320000
NLAB = 1440
NC = 2
NS = 16
HALFN = 5008
ARTF = 320
CHUNK = 32
CPT = 640
GC = 8
E_PAD = CHUNK * CPT * NS
LANES = 16
RB = 32
SENT = -1


def _seg_sum_body(h_hbm, src_hbm, dst_hbm, out_hbm,
                  eidx_v, rowsab_v, table_sh, agg_sh,
                  semg0, semg1):
    rows_v = rowsab_v.at[pl.ds(0, RB)]
    rowsb_v = rowsab_v.at[pl.ds(RB, RB)]
    cid = lax.axis_index("c")
    sid = lax.axis_index("s")
    base = cid * HALFN

    def _zb(t, carry):
        rows_v[t // (D // LANES),
               pl.ds((t % (D // LANES)) * LANES, LANES)] = (
            jnp.zeros((LANES,), jnp.float32))
        return carry
    lax.fori_loop(0, RB * (D // LANES), _zb, 0)

    @pl.when(sid < NS - 1)
    def _():
        for k in range(ARTF // RB):
            pltpu.sync_copy(rows_v,
                            agg_sh.at[pl.ds(sid * ARTF + k * RB, RB)])

    @pl.when(sid == NS - 1)
    def _():
        for k in range(6):
            pltpu.sync_copy(rows_v, agg_sh.at[pl.ds(4800 + k * RB, RB)])
        pltpu.sync_copy(rows_v.at[pl.ds(0, 16)], agg_sh.at[pl.ds(4992, 16)])

    def _stage_rows(r0, nk):
        bufs = (rows_v, rowsb_v)
        pltpu.async_copy(h_hbm.at[pl.ds(r0, RB)], bufs[0], semg0)
        for k in range(nk):
            b = bufs[k % 2]
            sl = pl.ds(r0 + k * RB, RB)
            pltpu.make_async_copy(h_hbm.at[sl], b, semg0).wait()
            if k + 1 < nk:
                pltpu.async_copy(h_hbm.at[pl.ds(r0 + (k + 1) * RB, RB)],
                                 bufs[(k + 1) % 2], semg0)
            pltpu.sync_copy(b, table_sh.at[sl])

    @pl.when(sid < NS - 1)
    def _():
        _stage_rows(sid * 640, 640 // RB)

    @pl.when(sid == NS - 1)
    def _():
        _stage_rows(9600, 400 // RB)
        tl = pl.ds(9984, 16)
        pltpu.sync_copy(h_hbm.at[tl], rows_v.at[pl.ds(0, 16)])
        pltpu.sync_copy(rows_v.at[pl.ds(0, 16)], table_sh.at[tl])

    plsc.subcore_barrier()

    def _group(j, carry):
        pltpu.sync_copy(src_hbm.at[pl.ds(sid * CPT + j * GC, GC)],
                        eidx_v.at[pl.ds(0, GC)])
        pltpu.sync_copy(dst_hbm.at[pl.ds(sid * CPT + j * GC, GC)],
                        eidx_v.at[pl.ds(GC, GC)])
        for t in range(GC * (CHUNK // LANES)):
            row, col = t // (CHUNK // LANES), (t % (CHUNK // LANES)) * LANES
            vs = eidx_v[row, pl.ds(col, LANES)]
            vd = eidx_v[GC + row, pl.ds(col, LANES)]
            loc = vd - base
            ok = jnp.logical_and(loc >= 0, loc < HALFN)
            eidx_v[GC + row, pl.ds(col, LANES)] = jnp.where(ok, loc, SENT)
            eidx_v[row, pl.ds(col, LANES)] = jnp.where(ok, vs, SENT)

        def _gather(cg, buf, sem):
            return pltpu.async_copy(
                table_sh.at[plsc.Indices(eidx_v.at[cg], ignored_value=SENT)],
                buf, sem)

        def _scatter(cg, buf):
            pltpu.sync_copy(
                buf, agg_sh.at[plsc.Indices(eidx_v.at[GC + cg],
                                            ignored_value=SENT)],
                add=True)

        _gather(0, rows_v, semg0)
        for p in range(GC // 2):
            pltpu.make_async_copy(table_sh, rows_v, semg0).wait()
            if 2 * p + 1 < GC:
                _gather(2 * p + 1, rowsb_v, semg1)
            _scatter(2 * p, rows_v)
            if 2 * p + 1 < GC:
                pltpu.make_async_copy(table_sh, rowsb_v, semg1).wait()
                if 2 * p + 2 < GC:
                    _gather(2 * p + 2, rows_v, semg0)
                _scatter(2 * p + 1, rowsb_v)
        return carry
    lax.fori_loop(0, CPT // GC, _group, 0)
    plsc.subcore_barrier()

    @pl.when(sid < NS - 1)
    def _():
        for k in range(ARTF // RB):
            sl = sid * ARTF + k * RB
            pltpu.sync_copy(agg_sh.at[pl.ds(sl, RB)],
                            out_hbm.at[pl.ds(base + sl, RB)])

    @pl.when(sid == NS - 1)
    def _():
        for k in range(6):
            sl = 4800 + k * RB
            pltpu.sync_copy(agg_sh.at[pl.ds(sl, RB)],
                            out_hbm.at[pl.ds(base + sl, RB)])
        pltpu.sync_copy(agg_sh.at[pl.ds(4992, 16)],
                        out_hbm.at[pl.ds(base + 4992, 16)])


@functools.cache
def _seg_sum():
    mesh = plsc.VectorSubcoreMesh(
        core_axis_name="c", subcore_axis_name="s",
        num_cores=NC, num_subcores=NS)
    return pl.kernel(
        _seg_sum_body,
        out_type=jax.ShapeDtypeStruct((2 * HALFN, D), jnp.float32),
        mesh=mesh,
        scratch_types=[
            pltpu.VMEM((2 * GC, CHUNK), jnp.int32),
            pltpu.VMEM((2 * RB, D), jnp.float32),
            pltpu.VMEM_SHARED((N, D), jnp.float32),
            pltpu.VMEM_SHARED((HALFN, D), jnp.float32),
            pltpu.SemaphoreType.DMA,
            pltpu.SemaphoreType.DMA,
        ],
    )


BR = 1000


def _combine_body(a, h, w1, w2, b, o):
    acc = jnp.dot(a[...], w1[...], preferred_element_type=jnp.float32)
    acc = acc + jnp.dot(h[...], w2[...], preferred_element_type=jnp.float32)
    o[...] = jnp.maximum(acc + b[...], 0.0)


def _combine(a, h, w1, w2, b):
    return pl.pallas_call(
        _combine_body,
        grid=(N // BR,),
        in_specs=[
            pl.BlockSpec((BR, D), lambda i: (i, 0)),
            pl.BlockSpec((BR, D), lambda i: (i, 0)),
            pl.BlockSpec((D, D), lambda i: (0, 0)),
            pl.BlockSpec((D, D), lambda i: (0, 0)),
            pl.BlockSpec((1, D), lambda i: (0, 0)),
        ],
        out_specs=pl.BlockSpec((BR, D), lambda i: (i, 0)),
        out_shape=jax.ShapeDtypeStruct((N, D), jnp.float32),
    )(a, h, w1, w2, b)


def _dense_body(h, wd, bd, o):
    o[...] = jnp.dot(h[...], wd[...], preferred_element_type=jnp.float32) + bd[...]


def _dense(h, wd, bd):
    return pl.pallas_call(
        _dense_body,
        grid=(N // BR,),
        in_specs=[
            pl.BlockSpec((BR, D), lambda i: (i, 0)),
            pl.BlockSpec((D, NLAB), lambda i: (0, 0)),
            pl.BlockSpec((1, NLAB), lambda i: (0, 0)),
        ],
        out_specs=pl.BlockSpec((BR, NLAB), lambda i: (i, 0)),
        out_shape=jax.ShapeDtypeStruct((N, NLAB), jnp.float32),
    )(h, wd, bd)


def kernel(x, edge_index,
           W1_0, W2_0, b_0,
           W1_1, W2_1, b_1,
           W1_2, W2_2, b_2,
           W1_3, W2_3, b_3,
           W1_4, W2_4, b_4,
           W1_5, W2_5, b_5,
           W1_6, W2_6, b_6,
           Wd, bd):
    src = edge_index[0]
    dst = edge_index[1]
    src_p = jnp.concatenate(
        [src, jnp.zeros((E_PAD - E,), jnp.int32)]).reshape(
            E_PAD // CHUNK, CHUNK)
    dst_p = jnp.concatenate(
        [dst, jnp.full((E_PAD - E,), N, jnp.int32)]).reshape(
            E_PAD // CHUNK, CHUNK)

    layers = [
        (W1_0, W2_0, b_0), (W1_1, W2_1, b_1), (W1_2, W2_2, b_2),
        (W1_3, W2_3, b_3), (W1_4, W2_4, b_4), (W1_5, W2_5, b_5),
        (W1_6, W2_6, b_6),
    ]
    h = x
    for w1, w2, b in layers:
        a = _seg_sum()(h, src_p, dst_p)
        h = _combine(a, h, w1, w2, b.reshape(1, D))
    return _dense(h, Wd, bd.reshape(1, NLAB))


# speedup vs baseline: 1.2714x; 1.0104x over previous
import functools

import jax
import jax.numpy as jnp
from jax import lax
from jax.experimental import pallas as pl
from jax.experimental.pallas import tpu as pltpu
from jax.experimental.pallas import tpu_sc as plsc

N = 10000
D = 128
E = 320000
NLAB = 1440
NC = 2
NS = 16
HALFN = 5008
ARTF = 320
CHUNK = 32
CPT = 640
GC = 8
E_PAD = CHUNK * CPT * NS
LANES = 16
RB = 32
SENT = -1


def _seg_sum_body(h_hbm, src_hbm, dst_hbm, out_hbm,
                  eidx_v, rowsab_v, table_sh, agg_sh,
                  semg0, semg1):
    rows_v = rowsab_v.at[pl.ds(0, RB)]
    rowsb_v = rowsab_v.at[pl.ds(RB, RB)]
    cid = lax.axis_index("c")
    sid = lax.axis_index("s")
    base = cid * HALFN

    def _zb(t, carry):
        rows_v[t // (D // LANES),
               pl.ds((t % (D // LANES)) * LANES, LANES)] = (
            jnp.zeros((LANES,), jnp.float32))
        return carry
    lax.fori_loop(0, RB * (D // LANES), _zb, 0)

    @pl.when(sid < NS - 1)
    def _():
        for k in range(ARTF // RB):
            pltpu.async_copy(
                rows_v, agg_sh.at[pl.ds(sid * ARTF + k * RB, RB)], semg1)
        for k in range(ARTF // RB):
            pltpu.make_async_copy(
                rows_v, agg_sh.at[pl.ds(sid * ARTF + k * RB, RB)],
                semg1).wait()

    @pl.when(sid == NS - 1)
    def _():
        for k in range(6):
            pltpu.async_copy(rows_v, agg_sh.at[pl.ds(4800 + k * RB, RB)],
                             semg1)
        pltpu.async_copy(rows_v.at[pl.ds(0, 16)],
                         agg_sh.at[pl.ds(4992, 16)], semg1)
        for k in range(6):
            pltpu.make_async_copy(
                rows_v, agg_sh.at[pl.ds(4800 + k * RB, RB)], semg1).wait()
        pltpu.make_async_copy(rows_v.at[pl.ds(0, 16)],
                              agg_sh.at[pl.ds(4992, 16)], semg1).wait()

    def _stage_rows(r0, nk):
        bufs = (rows_v, rowsb_v)
        pltpu.async_copy(h_hbm.at[pl.ds(r0, RB)], bufs[0], semg0)
        for k in range(nk):
            b = bufs[k % 2]
            sl = pl.ds(r0 + k * RB, RB)
            pltpu.make_async_copy(h_hbm.at[sl], b, semg0).wait()
            if k + 1 < nk:
                pltpu.async_copy(h_hbm.at[pl.ds(r0 + (k + 1) * RB, RB)],
                                 bufs[(k + 1) % 2], semg0)
            pltpu.sync_copy(b, table_sh.at[sl])

    @pl.when(sid < NS - 1)
    def _():
        _stage_rows(sid * 640, 640 // RB)

    @pl.when(sid == NS - 1)
    def _():
        _stage_rows(9600, 400 // RB)
        tl = pl.ds(9984, 16)
        pltpu.sync_copy(h_hbm.at[tl], rows_v.at[pl.ds(0, 16)])
        pltpu.sync_copy(rows_v.at[pl.ds(0, 16)], table_sh.at[tl])

    plsc.subcore_barrier()

    def _group(j, carry):
        pltpu.sync_copy(src_hbm.at[pl.ds(sid * CPT + j * GC, GC)],
                        eidx_v.at[pl.ds(0, GC)])
        pltpu.sync_copy(dst_hbm.at[pl.ds(sid * CPT + j * GC, GC)],
                        eidx_v.at[pl.ds(GC, GC)])
        for t in range(GC * (CHUNK // LANES)):
            row, col = t // (CHUNK // LANES), (t % (CHUNK // LANES)) * LANES
            vs = eidx_v[row, pl.ds(col, LANES)]
            vd = eidx_v[GC + row, pl.ds(col, LANES)]
            loc = vd - base
            ok = jnp.logical_and(loc >= 0, loc < HALFN)
            eidx_v[GC + row, pl.ds(col, LANES)] = jnp.where(ok, loc, SENT)
            eidx_v[row, pl.ds(col, LANES)] = jnp.where(ok, vs, SENT)

        def _gather(cg, buf, sem):
            return pltpu.async_copy(
                table_sh.at[plsc.Indices(eidx_v.at[cg], ignored_value=SENT)],
                buf, sem)

        def _scatter(cg, buf):
            pltpu.sync_copy(
                buf, agg_sh.at[plsc.Indices(eidx_v.at[GC + cg],
                                            ignored_value=SENT)],
                add=True)

        _gather(0, rows_v, semg0)
        for p in range(GC // 2):
            pltpu.make_async_copy(table_sh, rows_v, semg0).wait()
            if 2 * p + 1 < GC:
                _gather(2 * p + 1, rowsb_v, semg1)
            _scatter(2 * p, rows_v)
            if 2 * p + 1 < GC:
                pltpu.make_async_copy(table_sh, rowsb_v, semg1).wait()
                if 2 * p + 2 < GC:
                    _gather(2 * p + 2, rows_v, semg0)
                _scatter(2 * p + 1, rowsb_v)
        return carry
    lax.fori_loop(0, CPT // GC, _group, 0)
    plsc.subcore_barrier()

    @pl.when(sid < NS - 1)
    def _():
        for k in range(ARTF // RB):
            sl = sid * ARTF + k * RB
            pltpu.async_copy(agg_sh.at[pl.ds(sl, RB)],
                             out_hbm.at[pl.ds(base + sl, RB)], semg0)
        for k in range(ARTF // RB):
            sl = sid * ARTF + k * RB
            pltpu.make_async_copy(agg_sh.at[pl.ds(sl, RB)],
                                  out_hbm.at[pl.ds(base + sl, RB)],
                                  semg0).wait()

    @pl.when(sid == NS - 1)
    def _():
        for k in range(6):
            sl = 4800 + k * RB
            pltpu.async_copy(agg_sh.at[pl.ds(sl, RB)],
                             out_hbm.at[pl.ds(base + sl, RB)], semg0)
        pltpu.async_copy(agg_sh.at[pl.ds(4992, 16)],
                         out_hbm.at[pl.ds(base + 4992, 16)], semg0)
        for k in range(6):
            sl = 4800 + k * RB
            pltpu.make_async_copy(agg_sh.at[pl.ds(sl, RB)],
                                  out_hbm.at[pl.ds(base + sl, RB)],
                                  semg0).wait()
        pltpu.make_async_copy(agg_sh.at[pl.ds(4992, 16)],
                              out_hbm.at[pl.ds(base + 4992, 16)],
                              semg0).wait()


@functools.cache
def _seg_sum():
    mesh = plsc.VectorSubcoreMesh(
        core_axis_name="c", subcore_axis_name="s",
        num_cores=NC, num_subcores=NS)
    return pl.kernel(
        _seg_sum_body,
        out_type=jax.ShapeDtypeStruct((2 * HALFN, D), jnp.float32),
        mesh=mesh,
        scratch_types=[
            pltpu.VMEM((2 * GC, CHUNK), jnp.int32),
            pltpu.VMEM((2 * RB, D), jnp.float32),
            pltpu.VMEM_SHARED((N, D), jnp.float32),
            pltpu.VMEM_SHARED((HALFN, D), jnp.float32),
            pltpu.SemaphoreType.DMA,
            pltpu.SemaphoreType.DMA,
        ],
    )


BR = 1000


def _combine_body(a, h, w1, w2, b, o):
    acc = jnp.dot(a[...], w1[...], preferred_element_type=jnp.float32)
    acc = acc + jnp.dot(h[...], w2[...], preferred_element_type=jnp.float32)
    o[...] = jnp.maximum(acc + b[...], 0.0)


def _combine(a, h, w1, w2, b):
    return pl.pallas_call(
        _combine_body,
        grid=(N // BR,),
        in_specs=[
            pl.BlockSpec((BR, D), lambda i: (i, 0)),
            pl.BlockSpec((BR, D), lambda i: (i, 0)),
            pl.BlockSpec((D, D), lambda i: (0, 0)),
            pl.BlockSpec((D, D), lambda i: (0, 0)),
            pl.BlockSpec((1, D), lambda i: (0, 0)),
        ],
        out_specs=pl.BlockSpec((BR, D), lambda i: (i, 0)),
        out_shape=jax.ShapeDtypeStruct((N, D), jnp.float32),
    )(a, h, w1, w2, b)


def _dense_body(h, wd, bd, o):
    o[...] = jnp.dot(h[...], wd[...], preferred_element_type=jnp.float32) + bd[...]


def _dense(h, wd, bd):
    return pl.pallas_call(
        _dense_body,
        grid=(N // BR,),
        in_specs=[
            pl.BlockSpec((BR, D), lambda i: (i, 0)),
            pl.BlockSpec((D, NLAB), lambda i: (0, 0)),
            pl.BlockSpec((1, NLAB), lambda i: (0, 0)),
        ],
        out_specs=pl.BlockSpec((BR, NLAB), lambda i: (i, 0)),
        out_shape=jax.ShapeDtypeStruct((N, NLAB), jnp.float32),
    )(h, wd, bd)


def kernel(x, edge_index,
           W1_0, W2_0, b_0,
           W1_1, W2_1, b_1,
           W1_2, W2_2, b_2,
           W1_3, W2_3, b_3,
           W1_4, W2_4, b_4,
           W1_5, W2_5, b_5,
           W1_6, W2_6, b_6,
           Wd, bd):
    src = edge_index[0]
    dst = edge_index[1]
    src_p = jnp.concatenate(
        [src, jnp.zeros((E_PAD - E,), jnp.int32)]).reshape(
            E_PAD // CHUNK, CHUNK)
    dst_p = jnp.concatenate(
        [dst, jnp.full((E_PAD - E,), N, jnp.int32)]).reshape(
            E_PAD // CHUNK, CHUNK)

    layers = [
        (W1_0, W2_0, b_0), (W1_1, W2_1, b_1), (W1_2, W2_2, b_2),
        (W1_3, W2_3, b_3), (W1_4, W2_4, b_4), (W1_5, W2_5, b_5),
        (W1_6, W2_6, b_6),
    ]
    h = x
    for w1, w2, b in layers:
        a = _seg_sum()(h, src_p, dst_p)
        h = _combine(a, h, w1, w2, b.reshape(1, D))
    return _dense(h, Wd, bd.reshape(1, NLAB))
